# Initial kernel scaffold; baseline (speedup 1.0000x reference)
import jax
import jax.numpy as jnp
from jax.experimental import pallas as pl


def kernel(x, edge_index, W1, b1, prelu_a, W2, b2):
    raise NotImplementedError("write your pallas kernel here")


# speedup vs baseline: 20.3189x; 20.3189x over previous
import functools

import jax
import jax.numpy as jnp
from jax import lax
from jax.experimental import pallas as pl
from jax.experimental.pallas import tpu as pltpu
from jax.experimental.pallas import tpu_sc as plsc

NC = 2
NS = 16
CH = 80


def _deg_kernel_body(n, rpt, rows_per_w):
    def body(dst_hbm, z16_hbm, ones_hbm, degp_hbm, didx, ones_v, zbuf, acc, sem):
        c = lax.axis_index("c")
        s = lax.axis_index("s")
        w = c * NS + s
        pltpu.sync_copy(dst_hbm.at[w], didx)
        pltpu.sync_copy(ones_hbm, ones_v)
        pltpu.sync_copy(z16_hbm, zbuf)
        pltpu.sync_copy(zbuf, acc.at[pl.ds(s * rpt, rpt)])
        plsc.subcore_barrier()

        @pl.loop(0, rows_per_w)
        def _(j):
            pltpu.sync_copy(ones_v, acc.at[didx.at[j]], add=True)

        plsc.subcore_barrier()
        pltpu.sync_copy(acc.at[pl.ds(s * rpt, rpt)], degp_hbm.at[c, s])

    return body


def _prop_edge_split_body(n, d, rpt, rows_per_w):
    zch = rpt // 5

    def body(tab_hbm, src_hbm, dst_hbm, zd_hbm, out_hbm,
             sidx, didx, rows, zbuf, acc, sem):
        c = lax.axis_index("c")
        s = lax.axis_index("s")
        w = c * NS + s
        pltpu.sync_copy(src_hbm.at[w], sidx)
        pltpu.sync_copy(dst_hbm.at[w], didx)
        pltpu.sync_copy(zd_hbm, zbuf)

        @pl.loop(0, 5)
        def _(k):
            pltpu.sync_copy(zbuf, acc.at[pl.ds(s * rpt + k * zch, zch)])

        plsc.subcore_barrier()

        @pl.loop(0, rows_per_w)
        def _(j):
            pltpu.async_copy(tab_hbm.at[sidx.at[j]], rows, sem).wait()
            pltpu.sync_copy(rows, acc.at[didx.at[j]], add=True)

        plsc.subcore_barrier()
        pltpu.sync_copy(acc.at[pl.ds(s * rpt, rpt)], out_hbm.at[c, s])

    return body


def _prop_col_split_body(n, d, rpt, rows_per_blk, nblk):
    zch = rpt // 5

    def body(taba_hbm, tabb_hbm, src_hbm, dst_hbm, zd_hbm, out_hbm,
             sidx, didx, rows, zbuf, acc, sem):
        c = lax.axis_index("c")
        s = lax.axis_index("s")
        pltpu.sync_copy(zd_hbm, zbuf)

        @pl.loop(0, 5)
        def _(k):
            pltpu.sync_copy(zbuf, acc.at[pl.ds(s * rpt + k * zch, zch)])

        plsc.subcore_barrier()

        def run(tab):
            @pl.loop(0, nblk)
            def _(h):
                pltpu.sync_copy(src_hbm.at[s * nblk + h], sidx)
                pltpu.sync_copy(dst_hbm.at[s * nblk + h], didx)

                @pl.loop(0, rows_per_blk)
                def _(j):
                    pltpu.async_copy(tab.at[sidx.at[j]], rows, sem).wait()
                    pltpu.sync_copy(rows, acc.at[didx.at[j]], add=True)

        @pl.when(c == 0)
        def _():
            run(taba_hbm)

        @pl.when(c == 1)
        def _():
            run(tabb_hbm)

        plsc.subcore_barrier()
        pltpu.sync_copy(acc.at[pl.ds(s * rpt, rpt)], out_hbm.at[c, s])

    return body


def _prep_body(degp_ref, x_ref, xs_ref, dinv_ref, dinv2_ref):
    deg = degp_ref[0, :, 0:1] + degp_ref[1, :, 0:1] + 1.0
    dinv = lax.rsqrt(deg)
    dinv_ref[...] = dinv
    dinv2_ref[...] = 1.0 / deg
    xs_ref[...] = x_ref[...] * dinv


def _main_body(p1_ref, x_ref, dinv_ref, dinv2_ref, w1_ref, b1_ref, a_ref,
               w2_ref, t_ref, tsa_ref, tsb_ref):
    d = x_ref.shape[1]
    dinv = dinv_ref[...]
    s1 = dinv * (p1_ref[0] + p1_ref[1]) + dinv2_ref[...] * x_ref[...]
    h = jnp.dot(s1, w1_ref[...], preferred_element_type=jnp.float32) + b1_ref[...]
    a = a_ref[0, 0]
    h = jnp.where(h >= 0, h, a * h)
    t = jnp.dot(h, w2_ref[...], preferred_element_type=jnp.float32)
    t_ref[...] = t
    ts = dinv * t
    tsa_ref[...] = ts[:, :d]
    tsb_ref[...] = ts[:, d:]


def _final_body(p2_ref, t_ref, dinv_ref, dinv2_ref, b2_ref, out_ref):
    agg = jnp.concatenate([p2_ref[0], p2_ref[1]], axis=1)
    out_ref[...] = dinv_ref[...] * agg + dinv2_ref[...] * t_ref[...] + b2_ref[...]


def kernel(x, edge_index, W1, b1, prelu_a, W2, b2):
    n, d_in = x.shape
    e = edge_index.shape[1]
    d_mid = W1.shape[1]
    d_out = W2.shape[1]
    dh = d_out // 2
    assert d_in == dh, (d_in, dh)
    nr = e // CH
    rpt = n // NS
    rows_w1 = nr // (NC * NS)

    src32 = edge_index[0].reshape(NC * NS, rows_w1, CH)
    dst32 = edge_index[1].reshape(NC * NS, rows_w1, CH)
    z16 = jnp.zeros((rpt, 16), jnp.float32)
    zd = jnp.zeros((rpt // 5, d_in), jnp.float32)
    ones16 = jnp.ones((CH, 16), jnp.float32)

    mesh = plsc.VectorSubcoreMesh(core_axis_name="c", subcore_axis_name="s")
    sc_params = pltpu.CompilerParams(use_tc_tiling_on_sc=False)

    deg_call = functools.partial(
        pl.kernel,
        out_type=jax.ShapeDtypeStruct((NC, NS, rpt, 16), jnp.float32),
        mesh=mesh,
        compiler_params=sc_params,
        scratch_types=[
            pltpu.VMEM((rows_w1, CH), jnp.int32),
            pltpu.VMEM((CH, 16), jnp.float32),
            pltpu.VMEM((rpt, 16), jnp.float32),
            pltpu.VMEM_SHARED((n, 16), jnp.float32),
            pltpu.SemaphoreType.DMA,
        ],
    )(_deg_kernel_body(n, rpt, rows_w1))
    degp = deg_call(dst32, z16, ones16).reshape(NC, n, 16)

    xs, dinv, dinv2 = pl.pallas_call(
        _prep_body,
        out_shape=[
            jax.ShapeDtypeStruct((n, d_in), jnp.float32),
            jax.ShapeDtypeStruct((n, 1), jnp.float32),
            jax.ShapeDtypeStruct((n, 1), jnp.float32),
        ],
    )(degp, x)

    prop1_call = functools.partial(
        pl.kernel,
        out_type=jax.ShapeDtypeStruct((NC, NS, rpt, d_in), jnp.float32),
        mesh=mesh,
        compiler_params=sc_params,
        scratch_types=[
            pltpu.VMEM((rows_w1, CH), jnp.int32),
            pltpu.VMEM((rows_w1, CH), jnp.int32),
            pltpu.VMEM((CH, d_in), jnp.float32),
            pltpu.VMEM((rpt // 5, d_in), jnp.float32),
            pltpu.VMEM_SHARED((n, d_in), jnp.float32),
            pltpu.SemaphoreType.DMA,
        ],
    )(_prop_edge_split_body(n, d_in, rpt, rows_w1))
    p1 = prop1_call(xs, src32, dst32, zd).reshape(NC, n, d_in)

    blk = 2000
    grid = n // blk
    t, tsa, tsb = pl.pallas_call(
        _main_body,
        grid=(grid,),
        in_specs=[
            pl.BlockSpec((NC, blk, d_in), lambda i: (0, i, 0)),
            pl.BlockSpec((blk, d_in), lambda i: (i, 0)),
            pl.BlockSpec((blk, 1), lambda i: (i, 0)),
            pl.BlockSpec((blk, 1), lambda i: (i, 0)),
            pl.BlockSpec((d_in, d_mid), lambda i: (0, 0)),
            pl.BlockSpec((1, d_mid), lambda i: (0, 0)),
            pl.BlockSpec((1, 1), lambda i: (0, 0)),
            pl.BlockSpec((d_mid, d_out), lambda i: (0, 0)),
        ],
        out_specs=[
            pl.BlockSpec((blk, d_out), lambda i: (i, 0)),
            pl.BlockSpec((blk, dh), lambda i: (i, 0)),
            pl.BlockSpec((blk, dh), lambda i: (i, 0)),
        ],
        out_shape=[
            jax.ShapeDtypeStruct((n, d_out), jnp.float32),
            jax.ShapeDtypeStruct((n, dh), jnp.float32),
            jax.ShapeDtypeStruct((n, dh), jnp.float32),
        ],
    )(p1, x, dinv, dinv2, W1, b1.reshape(1, d_mid), prelu_a.reshape(1, 1), W2)

    prop2_call = functools.partial(
        pl.kernel,
        out_type=jax.ShapeDtypeStruct((NC, NS, rpt, dh), jnp.float32),
        mesh=mesh,
        compiler_params=sc_params,
        scratch_types=[
            pltpu.VMEM((rows_w1, CH), jnp.int32),
            pltpu.VMEM((rows_w1, CH), jnp.int32),
            pltpu.VMEM((CH, dh), jnp.float32),
            pltpu.VMEM((rpt // 5, dh), jnp.float32),
            pltpu.VMEM_SHARED((n, dh), jnp.float32),
            pltpu.SemaphoreType.DMA,
        ],
    )(_prop_col_split_body(n, dh, rpt, rows_w1, NC))
    p2 = prop2_call(tsa, tsb, src32, dst32, zd).reshape(NC, n, dh)

    out = pl.pallas_call(
        _final_body,
        grid=(grid,),
        in_specs=[
            pl.BlockSpec((NC, blk, dh), lambda i: (0, i, 0)),
            pl.BlockSpec((blk, d_out), lambda i: (i, 0)),
            pl.BlockSpec((blk, 1), lambda i: (i, 0)),
            pl.BlockSpec((blk, 1), lambda i: (i, 0)),
            pl.BlockSpec((1, d_out), lambda i: (0, 0)),
        ],
        out_specs=pl.BlockSpec((blk, d_out), lambda i: (i, 0)),
        out_shape=jax.ShapeDtypeStruct((n, d_out), jnp.float32),
    )(p2, t, dinv, dinv2, b2.reshape(1, d_out))
    return out


# speedup vs baseline: 25.6842x; 1.2641x over previous
import functools

import jax
import jax.numpy as jnp
from jax import lax
from jax.experimental import pallas as pl
from jax.experimental.pallas import tpu as pltpu
from jax.experimental.pallas import tpu_sc as plsc

NC = 2
NS = 16
CH = 80


def _deg_kernel_body(n, rpt, rows_per_w):
    def body(dst_hbm, z16_hbm, ones_hbm, degp_hbm, didx, ones_v, zbuf, acc, sem):
        c = lax.axis_index("c")
        s = lax.axis_index("s")
        w = c * NS + s
        pltpu.sync_copy(dst_hbm.at[w], didx)
        pltpu.sync_copy(ones_hbm, ones_v)
        pltpu.sync_copy(z16_hbm, zbuf)
        pltpu.sync_copy(zbuf, acc.at[pl.ds(s * rpt, rpt)])
        plsc.subcore_barrier()

        @pl.loop(0, rows_per_w)
        def _(j):
            pltpu.sync_copy(ones_v, acc.at[didx.at[j]], add=True)

        plsc.subcore_barrier()
        pltpu.sync_copy(acc.at[pl.ds(s * rpt, rpt)], degp_hbm.at[c, s])

    return body


def _zero_acc(zd_hbm, rows0, acc, s, rpt):
    pltpu.sync_copy(zd_hbm, rows0)
    nfull = rpt // CH
    rem = rpt - nfull * CH

    @pl.loop(0, nfull)
    def _(k):
        pltpu.sync_copy(rows0, acc.at[pl.ds(s * rpt + k * CH, CH)])

    if rem:
        pltpu.sync_copy(rows0.at[pl.ds(0, rem)],
                        acc.at[pl.ds(s * rpt + nfull * CH, rem)])


def _prop_block(tab, sidx, didx, rows0, rows1, acc, sem, nrows):
    pltpu.async_copy(tab.at[sidx.at[0]], rows0, sem)

    @pl.loop(0, nrows // 2)
    def _(k):
        j = 2 * k
        pltpu.make_async_copy(tab.at[sidx.at[0]], rows0, sem).wait()
        pltpu.async_copy(tab.at[sidx.at[j + 1]], rows1, sem)
        pltpu.sync_copy(rows0, acc.at[didx.at[j]], add=True)
        pltpu.make_async_copy(tab.at[sidx.at[0]], rows1, sem).wait()

        @pl.when(j + 2 < nrows)
        def _():
            pltpu.async_copy(tab.at[sidx.at[j + 2]], rows0, sem)

        pltpu.sync_copy(rows1, acc.at[didx.at[j + 1]], add=True)

    if nrows % 2 == 1:
        pltpu.make_async_copy(tab.at[sidx.at[0]], rows0, sem).wait()
        pltpu.sync_copy(rows0, acc.at[didx.at[nrows - 1]], add=True)


def _prop_edge_split_body(n, d, rpt, rows_per_w):
    def body(tab_hbm, src_hbm, dst_hbm, zd_hbm, out_hbm,
             sidx, didx, rows0, rows1, acc, sem):
        c = lax.axis_index("c")
        s = lax.axis_index("s")
        w = c * NS + s
        pltpu.sync_copy(src_hbm.at[w], sidx)
        pltpu.sync_copy(dst_hbm.at[w], didx)
        _zero_acc(zd_hbm, rows0, acc, s, rpt)
        plsc.subcore_barrier()
        _prop_block(tab_hbm, sidx, didx, rows0, rows1, acc, sem, rows_per_w)
        plsc.subcore_barrier()
        pltpu.sync_copy(acc.at[pl.ds(s * rpt, rpt)], out_hbm.at[c, s])

    return body


def _prop_col_split_body(n, d, rpt, rows_per_blk, nblk):
    def body(taba_hbm, tabb_hbm, src_hbm, dst_hbm, zd_hbm, out_hbm,
             sidx, didx, rows0, rows1, acc, sem):
        c = lax.axis_index("c")
        s = lax.axis_index("s")
        _zero_acc(zd_hbm, rows0, acc, s, rpt)
        plsc.subcore_barrier()

        def run(tab):
            @pl.loop(0, nblk)
            def _(h):
                pltpu.sync_copy(src_hbm.at[s * nblk + h], sidx)
                pltpu.sync_copy(dst_hbm.at[s * nblk + h], didx)
                _prop_block(tab, sidx, didx, rows0, rows1, acc, sem,
                            rows_per_blk)

        @pl.when(c == 0)
        def _():
            run(taba_hbm)

        @pl.when(c == 1)
        def _():
            run(tabb_hbm)

        plsc.subcore_barrier()
        pltpu.sync_copy(acc.at[pl.ds(s * rpt, rpt)], out_hbm.at[c, s])

    return body


def _prep_body(degp_ref, x_ref, xs_ref, dinv_ref, dinv2_ref):
    deg = degp_ref[0, :, 0:1] + degp_ref[1, :, 0:1] + 1.0
    dinv = lax.rsqrt(deg)
    dinv_ref[...] = dinv
    dinv2_ref[...] = 1.0 / deg
    xs_ref[...] = x_ref[...] * dinv


def _main_body(p1_ref, x_ref, dinv_ref, dinv2_ref, w1_ref, b1_ref, a_ref,
               w2_ref, t_ref, tsa_ref, tsb_ref):
    d = x_ref.shape[1]
    dinv = dinv_ref[...]
    s1 = dinv * (p1_ref[0] + p1_ref[1]) + dinv2_ref[...] * x_ref[...]
    h = jnp.dot(s1, w1_ref[...], preferred_element_type=jnp.float32) + b1_ref[...]
    a = a_ref[0, 0]
    h = jnp.where(h >= 0, h, a * h)
    t = jnp.dot(h, w2_ref[...], preferred_element_type=jnp.float32)
    t_ref[...] = t
    ts = dinv * t
    tsa_ref[...] = ts[:, :d]
    tsb_ref[...] = ts[:, d:]


def _final_body(p2_ref, t_ref, dinv_ref, dinv2_ref, b2_ref, out_ref):
    agg = jnp.concatenate([p2_ref[0], p2_ref[1]], axis=1)
    out_ref[...] = dinv_ref[...] * agg + dinv2_ref[...] * t_ref[...] + b2_ref[...]


def kernel(x, edge_index, W1, b1, prelu_a, W2, b2):
    n, d_in = x.shape
    e = edge_index.shape[1]
    d_mid = W1.shape[1]
    d_out = W2.shape[1]
    dh = d_out // 2
    assert d_in == dh, (d_in, dh)
    nr = e // CH
    rpt = n // NS
    rows_w1 = nr // (NC * NS)

    src32 = edge_index[0].reshape(NC * NS, rows_w1, CH)
    dst32 = edge_index[1].reshape(NC * NS, rows_w1, CH)
    nblk2 = 5
    rows_p2 = nr // (NS * nblk2)
    srcp2 = edge_index[0].reshape(NS * nblk2, rows_p2, CH)
    dstp2 = edge_index[1].reshape(NS * nblk2, rows_p2, CH)
    z16 = jnp.zeros((rpt, 16), jnp.float32)
    zd = jnp.zeros((CH, d_in), jnp.float32)
    ones16 = jnp.ones((CH, 16), jnp.float32)

    mesh = plsc.VectorSubcoreMesh(core_axis_name="c", subcore_axis_name="s")
    sc_params = pltpu.CompilerParams(use_tc_tiling_on_sc=False)

    deg_call = functools.partial(
        pl.kernel,
        out_type=jax.ShapeDtypeStruct((NC, NS, rpt, 16), jnp.float32),
        mesh=mesh,
        compiler_params=sc_params,
        scratch_types=[
            pltpu.VMEM((rows_w1, CH), jnp.int32),
            pltpu.VMEM((CH, 16), jnp.float32),
            pltpu.VMEM((rpt, 16), jnp.float32),
            pltpu.VMEM_SHARED((n, 16), jnp.float32),
            pltpu.SemaphoreType.DMA,
        ],
    )(_deg_kernel_body(n, rpt, rows_w1))
    degp = deg_call(dst32, z16, ones16).reshape(NC, n, 16)

    xs, dinv, dinv2 = pl.pallas_call(
        _prep_body,
        out_shape=[
            jax.ShapeDtypeStruct((n, d_in), jnp.float32),
            jax.ShapeDtypeStruct((n, 1), jnp.float32),
            jax.ShapeDtypeStruct((n, 1), jnp.float32),
        ],
    )(degp, x)

    prop1_call = functools.partial(
        pl.kernel,
        out_type=jax.ShapeDtypeStruct((NC, NS, rpt, d_in), jnp.float32),
        mesh=mesh,
        compiler_params=sc_params,
        scratch_types=[
            pltpu.VMEM((rows_w1, CH), jnp.int32),
            pltpu.VMEM((rows_w1, CH), jnp.int32),
            pltpu.VMEM((CH, d_in), jnp.float32),
            pltpu.VMEM((CH, d_in), jnp.float32),
            pltpu.VMEM_SHARED((n, d_in), jnp.float32),
            pltpu.SemaphoreType.DMA,
        ],
    )(_prop_edge_split_body(n, d_in, rpt, rows_w1))
    p1 = prop1_call(xs, src32, dst32, zd).reshape(NC, n, d_in)

    blk = 2000
    grid = n // blk
    t, tsa, tsb = pl.pallas_call(
        _main_body,
        grid=(grid,),
        in_specs=[
            pl.BlockSpec((NC, blk, d_in), lambda i: (0, i, 0)),
            pl.BlockSpec((blk, d_in), lambda i: (i, 0)),
            pl.BlockSpec((blk, 1), lambda i: (i, 0)),
            pl.BlockSpec((blk, 1), lambda i: (i, 0)),
            pl.BlockSpec((d_in, d_mid), lambda i: (0, 0)),
            pl.BlockSpec((1, d_mid), lambda i: (0, 0)),
            pl.BlockSpec((1, 1), lambda i: (0, 0)),
            pl.BlockSpec((d_mid, d_out), lambda i: (0, 0)),
        ],
        out_specs=[
            pl.BlockSpec((blk, d_out), lambda i: (i, 0)),
            pl.BlockSpec((blk, dh), lambda i: (i, 0)),
            pl.BlockSpec((blk, dh), lambda i: (i, 0)),
        ],
        out_shape=[
            jax.ShapeDtypeStruct((n, d_out), jnp.float32),
            jax.ShapeDtypeStruct((n, dh), jnp.float32),
            jax.ShapeDtypeStruct((n, dh), jnp.float32),
        ],
    )(p1, x, dinv, dinv2, W1, b1.reshape(1, d_mid), prelu_a.reshape(1, 1), W2)

    prop2_call = functools.partial(
        pl.kernel,
        out_type=jax.ShapeDtypeStruct((NC, NS, rpt, dh), jnp.float32),
        mesh=mesh,
        compiler_params=sc_params,
        scratch_types=[
            pltpu.VMEM((rows_p2, CH), jnp.int32),
            pltpu.VMEM((rows_p2, CH), jnp.int32),
            pltpu.VMEM((CH, dh), jnp.float32),
            pltpu.VMEM((CH, dh), jnp.float32),
            pltpu.VMEM_SHARED((n, dh), jnp.float32),
            pltpu.SemaphoreType.DMA,
        ],
    )(_prop_col_split_body(n, dh, rpt, rows_p2, nblk2))
    p2 = prop2_call(tsa, tsb, srcp2, dstp2, zd).reshape(NC, n, dh)

    out = pl.pallas_call(
        _final_body,
        grid=(grid,),
        in_specs=[
            pl.BlockSpec((NC, blk, dh), lambda i: (0, i, 0)),
            pl.BlockSpec((blk, d_out), lambda i: (i, 0)),
            pl.BlockSpec((blk, 1), lambda i: (i, 0)),
            pl.BlockSpec((blk, 1), lambda i: (i, 0)),
            pl.BlockSpec((1, d_out), lambda i: (0, 0)),
        ],
        out_specs=pl.BlockSpec((blk, d_out), lambda i: (i, 0)),
        out_shape=jax.ShapeDtypeStruct((n, d_out), jnp.float32),
    )(p2, t, dinv, dinv2, b2.reshape(1, d_out))
    return out
